# baseline (speedup 1.0000x reference)
import jax
import jax.numpy as jnp
from jax import lax
from jax.experimental import pallas as pl
from jax.experimental.pallas import tpu as pltpu

_BM = 200
_DEPTH = 4


def _gcn_kernel(x_ref, w_ref, b_ref, adj_ref, o_ref, bufs, s_ref, sems):
    n = x_ref.shape[0]
    nblk = n // _BM

    s_ref[...] = jnp.dot(x_ref[...], w_ref[...],
                         preferred_element_type=jnp.float32)

    def start(idx, slot):
        pltpu.make_async_copy(
            adj_ref.at[pl.ds(idx * _BM, _BM), :], bufs.at[slot], sems.at[slot]
        ).start()

    for d in range(_DEPTH):
        start(d, d)

    def body(i, carry):
        slot = lax.rem(i, _DEPTH)
        pltpu.make_async_copy(
            adj_ref.at[pl.ds(i * _BM, _BM), :], bufs.at[slot], sems.at[slot]
        ).wait()
        p = jnp.dot(bufs[slot], s_ref[...], preferred_element_type=jnp.float32)
        o_ref[pl.ds(i * _BM, _BM), :] = jnp.maximum(p + b_ref[...], 0.0)

        @pl.when(i + _DEPTH < nblk)
        def _prefetch():
            start(i + _DEPTH, slot)

        return carry

    lax.fori_loop(0, nblk, body, 0)


def kernel(x, adj, W, b):
    n, nfeat = x.shape
    nout = W.shape[1]

    out = pl.pallas_call(
        _gcn_kernel,
        in_specs=[
            pl.BlockSpec(memory_space=pltpu.MemorySpace.VMEM),
            pl.BlockSpec(memory_space=pltpu.MemorySpace.VMEM),
            pl.BlockSpec(memory_space=pltpu.MemorySpace.VMEM),
            pl.BlockSpec(memory_space=pl.ANY),
        ],
        out_specs=pl.BlockSpec(memory_space=pltpu.MemorySpace.VMEM),
        out_shape=jax.ShapeDtypeStruct((n, nout), jnp.float32),
        scratch_shapes=[
            pltpu.VMEM((_DEPTH, _BM, n), jnp.float32),
            pltpu.VMEM((n, nout), jnp.float32),
            pltpu.SemaphoreType.DMA((_DEPTH,)),
        ],
    )(x, W, b.reshape(1, nout), adj)
    return out
